# Initial kernel scaffold; baseline (speedup 1.0000x reference)
import functools
import math

import jax
import jax.numpy as jnp
from jax import lax
from jax.experimental import pallas as pl
from jax.experimental.pallas import tpu as pltpu
from jax.experimental.pallas import tpu_sc as plsc

N = 10000
E = 320000
D = 128
OUT = 16

NC = 2
NS = 16
NW = NC * NS
EPW = E // NW
CH = 128
NCHUNK = (EPW + CH - 1) // CH
EPW_PAD = NCHUNK * CH
ROWS = 10016
RPT = ROWS // NS


def _make_sc_agg(W):
  mesh = plsc.VectorSubcoreMesh(core_axis_name="c", subcore_axis_name="s")

  @functools.partial(
      pl.kernel,
      out_type=jax.ShapeDtypeStruct((NC, ROWS, W), jnp.float32),
      mesh=mesh,
      scratch_types=[
          pltpu.VMEM((NCHUNK, CH), jnp.int32),
          pltpu.VMEM((NCHUNK, CH), jnp.int32),
          pltpu.VMEM((CH, W), jnp.float32),
          pltpu.VMEM_SHARED((ROWS, W), jnp.float32),
          pltpu.SemaphoreType.DMA,
      ],
  )
  def agg(xe_hbm, src_hbm, dst_hbm, zeros_hbm, out_hbm,
          src_v, dst_v, rows_v, acc_sh, sem):
    c = lax.axis_index("c")
    s = lax.axis_index("s")
    pltpu.sync_copy(zeros_hbm, acc_sh.at[pl.ds(s * RPT, RPT)])
    pltpu.sync_copy(src_hbm.at[c, s], src_v)
    pltpu.sync_copy(dst_hbm.at[c, s], dst_v)
    plsc.subcore_barrier()

    def body(j, carry):
      pltpu.async_copy(xe_hbm.at[src_v.at[j]], rows_v, sem).wait()
      pltpu.sync_copy(rows_v, acc_sh.at[dst_v.at[j]], add=True)
      return carry

    lax.fori_loop(0, NCHUNK, body, 0, unroll=False)
    plsc.subcore_barrier()
    pltpu.sync_copy(acc_sh.at[pl.ds(s * RPT, RPT)],
                    out_hbm.at[c, pl.ds(s * RPT, RPT)])

  return agg


_sc_agg_wide = _make_sc_agg(D + 16)
_sc_agg = _make_sc_agg(D)


_INV_SQRT2 = 1.0 / math.sqrt(2.0)


def _gelu(h):
  return 0.5 * h * (1.0 + lax.erf(h * _INV_SQRT2))


def _tc_layer_body(a0, a1, deg, x, wlt, bl, wrt, g, be, o):
  r = 1.0 / jnp.maximum(deg[...], 1.0)
  mean = (a0[...] + a1[...]) * r
  h = (jnp.dot(mean, wlt[...], preferred_element_type=jnp.float32)
       + jnp.dot(x[...], wrt[...], preferred_element_type=jnp.float32)
       + bl[...])
  mu = jnp.mean(h, axis=-1, keepdims=True)
  var = jnp.mean((h - mu) ** 2, axis=-1, keepdims=True)
  h = (h - mu) / jnp.sqrt(var + 1e-5) * g[...] + be[...]
  o[...] = _gelu(h) + x[...]


BR = 400


def _tc_layer(a0, a1, deg, x, wlt, bl, wrt, g, be):
  grid = (N // BR,)
  return pl.pallas_call(
      _tc_layer_body,
      grid=grid,
      in_specs=[
          pl.BlockSpec((BR, D), lambda i: (i, 0)),
          pl.BlockSpec((BR, D), lambda i: (i, 0)),
          pl.BlockSpec((BR, 1), lambda i: (i, 0)),
          pl.BlockSpec((BR, D), lambda i: (i, 0)),
          pl.BlockSpec((D, D), lambda i: (0, 0)),
          pl.BlockSpec((1, D), lambda i: (0, 0)),
          pl.BlockSpec((D, D), lambda i: (0, 0)),
          pl.BlockSpec((1, D), lambda i: (0, 0)),
          pl.BlockSpec((1, D), lambda i: (0, 0)),
      ],
      out_specs=pl.BlockSpec((BR, D), lambda i: (i, 0)),
      out_shape=jax.ShapeDtypeStruct((N, D), jnp.float32),
  )(a0, a1, deg, x, wlt, bl, wrt, g, be)


def _tc_head_body(x, w1t, b1, w2t, b2, o):
  h = jnp.dot(x[...], w1t[...], preferred_element_type=jnp.float32) + b1[...]
  h = _gelu(h)
  o[...] = jnp.dot(h, w2t[...], preferred_element_type=jnp.float32) + b2[...]


def _tc_head(x, w1t, b1, w2t, b2):
  grid = (N // BR,)
  return pl.pallas_call(
      _tc_head_body,
      grid=grid,
      in_specs=[
          pl.BlockSpec((BR, D), lambda i: (i, 0)),
          pl.BlockSpec((D, D), lambda i: (0, 0)),
          pl.BlockSpec((1, D), lambda i: (0, 0)),
          pl.BlockSpec((D, OUT), lambda i: (0, 0)),
          pl.BlockSpec((1, OUT), lambda i: (0, 0)),
      ],
      out_specs=pl.BlockSpec((BR, OUT), lambda i: (i, 0)),
      out_shape=jax.ShapeDtypeStruct((N, OUT), jnp.float32),
  )(x, w1t, b1, w2t, b2)


@jax.jit
def kernel(x, edge_index, Wl0, bl0, Wr0, g0, be0, Wl1, bl1, Wr1, g1, be1,
           Wl2, bl2, Wr2, g2, be2, pW1, pb1, pW2, pb2):
  src = edge_index[0].astype(jnp.int32)
  dst = edge_index[1].astype(jnp.int32)
  srcw = jnp.pad(src.reshape(NW, EPW), ((0, 0), (0, EPW_PAD - EPW)))
  dstw = jnp.pad(dst.reshape(NW, EPW), ((0, 0), (0, EPW_PAD - EPW)),
                 constant_values=N)
  src3 = srcw.reshape(NC, NS, NCHUNK, CH)
  dst3 = dstw.reshape(NC, NS, NCHUNK, CH)

  WIDE = D + 16
  zeros_wide = jnp.zeros((RPT, WIDE), jnp.float32)
  zeros_d = jnp.zeros((RPT, D), jnp.float32)

  params = (Wl0, bl0, Wr0, g0, be0, Wl1, bl1, Wr1, g1, be1,
            Wl2, bl2, Wr2, g2, be2)

  deg = None
  h = x
  for i in range(3):
    Wl, bl, Wr, g, be = params[5 * i:5 * i + 5]
    if i == 0:
      ones_col = jnp.ones((N, 1), jnp.float32)
      xe = jnp.concatenate(
          [h, ones_col, jnp.zeros((N, WIDE - D - 1), jnp.float32)], axis=1)
      acc = _sc_agg_wide(xe, src3, dst3, zeros_wide)
      deg = (acc[0, :N, D] + acc[1, :N, D]).reshape(N, 1)
      a0 = acc[0, :N, :D]
      a1 = acc[1, :N, :D]
    else:
      acc = _sc_agg(h, src3, dst3, zeros_d)
      a0 = acc[0, :N, :]
      a1 = acc[1, :N, :]
    h = _tc_layer(a0, a1, deg, h, Wl.T, bl.reshape(1, D), Wr.T,
                  g.reshape(1, D), be.reshape(1, D))

  return _tc_head(h, pW1.T, pb1.reshape(1, D), pW2.T, pb2.reshape(1, OUT))


# speedup vs baseline: 4.8373x; 4.8373x over previous
import functools
import math

import jax
import jax.numpy as jnp
from jax import lax
from jax.experimental import pallas as pl
from jax.experimental.pallas import tpu as pltpu
from jax.experimental.pallas import tpu_sc as plsc

N = 10000
E = 320000
D = 128
OUT = 16

NC = 2
NS = 16
NW = NC * NS
EPW = E // NW
CH = 128
NCHUNK = (EPW + CH - 1) // CH
EPW_PAD = NCHUNK * CH
ROWS = 10112
RPT = ROWS // NS


@functools.lru_cache(maxsize=None)
def _make_sc_agg(W):
  mesh = plsc.VectorSubcoreMesh(core_axis_name="c", subcore_axis_name="s",
                                num_cores=NC, num_subcores=NS)

  @functools.partial(
      pl.kernel,
      out_type=jax.ShapeDtypeStruct((NC, ROWS, W), jnp.float32),
      mesh=mesh,
      scratch_types=[
          pltpu.VMEM((NCHUNK, CH), jnp.int32),
          pltpu.VMEM((NCHUNK, CH), jnp.int32),
          pltpu.VMEM((CH, W), jnp.float32),
          pltpu.VMEM_SHARED((ROWS, W), jnp.float32),
          pltpu.SemaphoreType.DMA,
      ],
  )
  def agg(xe_hbm, src_hbm, dst_hbm, zeros_hbm, out_hbm,
          src_v, dst_v, rows_v, acc_sh, sem):
    c = lax.axis_index("c")
    s = lax.axis_index("s")
    pltpu.sync_copy(zeros_hbm, acc_sh.at[pl.ds(s * RPT, RPT)])
    pltpu.sync_copy(src_hbm.at[c, s], src_v)
    pltpu.sync_copy(dst_hbm.at[c, s], dst_v)
    plsc.subcore_barrier()

    def body(j, carry):
      pltpu.async_copy(xe_hbm.at[src_v.at[j]], rows_v, sem).wait()
      pltpu.sync_copy(rows_v, acc_sh.at[dst_v.at[j]], add=True)
      return carry

    lax.fori_loop(0, NCHUNK, body, 0, unroll=False)
    plsc.subcore_barrier()
    pltpu.sync_copy(acc_sh.at[pl.ds(s * RPT, RPT)],
                    out_hbm.at[c, pl.ds(s * RPT, RPT)])

  return agg


@functools.lru_cache(maxsize=None)
def _make_sc_deg():
  W = D
  mesh = plsc.VectorSubcoreMesh(core_axis_name="c", subcore_axis_name="s",
                                num_cores=NC, num_subcores=NS)

  @functools.partial(
      pl.kernel,
      out_type=jax.ShapeDtypeStruct((NC, ROWS, W), jnp.float32),
      mesh=mesh,
      scratch_types=[
          pltpu.VMEM((NCHUNK, CH), jnp.int32),
          pltpu.VMEM((CH, W), jnp.float32),
          pltpu.VMEM_SHARED((ROWS, W), jnp.float32),
      ],
  )
  def degk(ones_hbm, dst_hbm, zeros_hbm, out_hbm, dst_v, ones_v, acc_sh):
    c = lax.axis_index("c")
    s = lax.axis_index("s")
    pltpu.sync_copy(zeros_hbm, acc_sh.at[pl.ds(s * RPT, RPT)])
    pltpu.sync_copy(dst_hbm.at[c, s], dst_v)
    pltpu.sync_copy(ones_hbm, ones_v)
    plsc.subcore_barrier()

    def body(j, carry):
      pltpu.sync_copy(ones_v, acc_sh.at[dst_v.at[j]], add=True)
      return carry

    lax.fori_loop(0, NCHUNK, body, 0, unroll=False)
    plsc.subcore_barrier()
    pltpu.sync_copy(acc_sh.at[pl.ds(s * RPT, RPT)],
                    out_hbm.at[c, pl.ds(s * RPT, RPT)])

  return degk


_INV_SQRT2 = 1.0 / math.sqrt(2.0)


def _gelu(h):
  return 0.5 * h * (1.0 + lax.erf(h * _INV_SQRT2))


def _tc_layer_body(a0, a1, deg, x, wlt, bl, wrt, g, be, o):
  r = 1.0 / jnp.maximum(deg[...], 1.0)
  mean = (a0[...] + a1[...]) * r
  h = (jnp.dot(mean, wlt[...], preferred_element_type=jnp.float32)
       + jnp.dot(x[...], wrt[...], preferred_element_type=jnp.float32)
       + bl[...])
  mu = jnp.mean(h, axis=-1, keepdims=True)
  var = jnp.mean((h - mu) ** 2, axis=-1, keepdims=True)
  h = (h - mu) / jnp.sqrt(var + 1e-5) * g[...] + be[...]
  o[...] = _gelu(h) + x[...]


BR = 400


def _tc_layer(a0, a1, deg, x, wlt, bl, wrt, g, be):
  grid = (N // BR,)
  return pl.pallas_call(
      _tc_layer_body,
      grid=grid,
      in_specs=[
          pl.BlockSpec((BR, D), lambda i: (i, 0)),
          pl.BlockSpec((BR, D), lambda i: (i, 0)),
          pl.BlockSpec((BR, 1), lambda i: (i, 0)),
          pl.BlockSpec((BR, D), lambda i: (i, 0)),
          pl.BlockSpec((D, D), lambda i: (0, 0)),
          pl.BlockSpec((1, D), lambda i: (0, 0)),
          pl.BlockSpec((D, D), lambda i: (0, 0)),
          pl.BlockSpec((1, D), lambda i: (0, 0)),
          pl.BlockSpec((1, D), lambda i: (0, 0)),
      ],
      out_specs=pl.BlockSpec((BR, D), lambda i: (i, 0)),
      out_shape=jax.ShapeDtypeStruct((N, D), jnp.float32),
  )(a0, a1, deg, x, wlt, bl, wrt, g, be)


def _tc_head_body(x, w1t, b1, w2t, b2, o):
  h = jnp.dot(x[...], w1t[...], preferred_element_type=jnp.float32) + b1[...]
  h = _gelu(h)
  o[...] = jnp.dot(h, w2t[...], preferred_element_type=jnp.float32) + b2[...]


def _tc_head(x, w1t, b1, w2t, b2):
  grid = (N // BR,)
  return pl.pallas_call(
      _tc_head_body,
      grid=grid,
      in_specs=[
          pl.BlockSpec((BR, D), lambda i: (i, 0)),
          pl.BlockSpec((D, D), lambda i: (0, 0)),
          pl.BlockSpec((1, D), lambda i: (0, 0)),
          pl.BlockSpec((D, OUT), lambda i: (0, 0)),
          pl.BlockSpec((1, OUT), lambda i: (0, 0)),
      ],
      out_specs=pl.BlockSpec((BR, OUT), lambda i: (i, 0)),
      out_shape=jax.ShapeDtypeStruct((N, OUT), jnp.float32),
  )(x, w1t, b1, w2t, b2)


@jax.jit
def kernel(x, edge_index, Wl0, bl0, Wr0, g0, be0, Wl1, bl1, Wr1, g1, be1,
           Wl2, bl2, Wr2, g2, be2, pW1, pb1, pW2, pb2):
  src = edge_index[0].astype(jnp.int32)
  dst = edge_index[1].astype(jnp.int32)
  srcw = jnp.pad(src.reshape(NW, EPW), ((0, 0), (0, EPW_PAD - EPW)))
  dstw = jnp.pad(dst.reshape(NW, EPW), ((0, 0), (0, EPW_PAD - EPW)),
                 constant_values=N)
  src3 = srcw.reshape(NC, NS, NCHUNK, CH)
  dst3 = dstw.reshape(NC, NS, NCHUNK, CH)

  zeros_d = jnp.zeros((RPT, D), jnp.float32)
  ones_d = jnp.ones((CH, D), jnp.float32)

  params = (Wl0, bl0, Wr0, g0, be0, Wl1, bl1, Wr1, g1, be1,
            Wl2, bl2, Wr2, g2, be2)

  dacc = _make_sc_deg()(ones_d, dst3, zeros_d)
  deg = (dacc[0, :N, 0] + dacc[1, :N, 0]).reshape(N, 1)

  h = x
  for i in range(3):
    Wl, bl, Wr, g, be = params[5 * i:5 * i + 5]
    acc = _make_sc_agg(D)(h, src3, dst3, zeros_d)
    a0 = acc[0, :N, :]
    a1 = acc[1, :N, :]
    h = _tc_layer(a0, a1, deg, h, Wl.T, bl.reshape(1, D), Wr.T,
                  g.reshape(1, D), be.reshape(1, D))

  return _tc_head(h, pW1.T, pb1.reshape(1, D), pW2.T, pb2.reshape(1, OUT))
